# Initial kernel scaffold; baseline (speedup 1.0000x reference)
import jax
import jax.numpy as jnp
from jax.experimental import pallas as pl


def kernel(feats, xyz, W1, b1, W2, b2):
    raise NotImplementedError("write your pallas kernel here")


# speedup vs baseline: 37.5797x; 37.5797x over previous
import functools

import jax
import jax.numpy as jnp
from jax import lax
from jax.experimental import pallas as pl
from jax.experimental.pallas import tpu as pltpu
from jax.experimental.pallas import tpu_sc as plsc

KNBR = 8
WIN = 1


def _precompute_pq(X, Wcat, bias, M, H):
    MB = 1024
    Kp = X.shape[1]

    def body(x_ref, w_ref, b_ref, p_ref, q_ref):
        y = jnp.dot(x_ref[...], w_ref[...],
                    preferred_element_type=jnp.float32) + b_ref[0:1, :]
        p_ref[...] = y[:, :H]
        q_ref[...] = y[:, H:]

    return pl.pallas_call(
        body,
        grid=(M // MB,),
        in_specs=[
            pl.BlockSpec((MB, Kp), lambda i: (i, 0)),
            pl.BlockSpec((Kp, 2 * H), lambda i: (0, 0)),
            pl.BlockSpec((8, 2 * H), lambda i: (0, 0)),
        ],
        out_specs=[
            pl.BlockSpec((MB, H), lambda i: (i, 0)),
            pl.BlockSpec((MB, H), lambda i: (i, 0)),
        ],
        out_shape=[
            jax.ShapeDtypeStruct((M, H), jnp.float32),
            jax.ShapeDtypeStruct((M, H), jnp.float32),
        ],
    )(X, Wcat, bias)


def _knn_indices(xyz_q, xyz_c, B, T, N):
    NB = 256
    F = 3
    c0max = T - F
    INF = 3e38
    BIG = 1e30

    def body(q_ref, c0_ref, c1_ref, c2_ref, o_ref):
        b = pl.program_id(0)
        t = pl.program_id(1)
        c0 = jnp.minimum(jnp.maximum(t - WIN, 0), c0max)
        q = q_ref[0, 0]
        qn = jnp.sum(q * q, axis=1)
        tiles = []
        for k, c_ref in enumerate((c0_ref, c1_ref, c2_ref)):
            c = c_ref[0, 0]
            cn = jnp.sum(c * c, axis=0)
            dot = jnp.dot(q, c, preferred_element_type=jnp.float32)
            valid = jnp.abs(c0 + k - t) <= WIN
            pen = jnp.where(valid, 0.0, BIG).astype(jnp.float32)
            tiles.append(qn[:, None] + cn[None, :] - 2.0 * dot + pen)
        d2 = jnp.concatenate(tiles, axis=1)
        ji = lax.broadcasted_iota(jnp.int32, (NB, F * N), 1)
        picks = []
        for _ in range(KNBR):
            m = jnp.min(d2, axis=1, keepdims=True)
            idx = jnp.min(jnp.where(d2 == m, ji, jnp.int32(F * N)), axis=1)
            picks.append(idx)
            d2 = jnp.where(ji == idx[:, None], INF, d2)
        base = (b * T + c0) * N
        o_ref[0, 0] = jnp.stack(picks, axis=1) + base

    def cmap(k):
        return lambda b, t, i: (b, jnp.minimum(jnp.maximum(t - WIN, 0), c0max) + k, 0, 0)

    return pl.pallas_call(
        body,
        grid=(B, T, N // NB),
        in_specs=[
            pl.BlockSpec((1, 1, NB, 8), lambda b, t, i: (b, t, i, 0)),
            pl.BlockSpec((1, 1, 8, N), cmap(0)),
            pl.BlockSpec((1, 1, 8, N), cmap(1)),
            pl.BlockSpec((1, 1, 8, N), cmap(2)),
        ],
        out_specs=pl.BlockSpec((1, 1, NB, KNBR), lambda b, t, i: (b, t, i, 0)),
        out_shape=jax.ShapeDtypeStruct((B, T, N, KNBR), jnp.int32),
    )(xyz_q, xyz_c, xyz_c, xyz_c)


def _gather_rows(table, idx):
    NE = idx.shape[0]
    H = table.shape[1]
    info = plsc.get_sparse_core_info()
    NW = info.num_cores * info.num_subcores
    NC = info.num_cores
    per_w = NE // NW
    CH = 128
    nchunk = per_w // CH
    mesh = plsc.VectorSubcoreMesh(core_axis_name="c", subcore_axis_name="s")

    @functools.partial(
        pl.kernel,
        mesh=mesh,
        out_type=jax.ShapeDtypeStruct((NE, H), jnp.float32),
        scratch_types=[
            pltpu.VMEM((CH,), jnp.int32),
            pltpu.VMEM((CH, H), jnp.float32),
            pltpu.SemaphoreType.DMA,
        ],
    )
    def gk(idx_hbm, table_hbm, out_hbm, idx_v, rows_v, sem):
        wid = lax.axis_index("s") * NC + lax.axis_index("c")
        base = wid * per_w

        def chunk(c, carry):
            off = base + c * CH
            pltpu.sync_copy(idx_hbm.at[pl.ds(off, CH)], idx_v)
            pltpu.async_copy(table_hbm.at[idx_v], rows_v, sem).wait()
            pltpu.sync_copy(rows_v, out_hbm.at[pl.ds(off, CH)])
            return carry

        lax.fori_loop(0, nchunk, chunk, jnp.int32(0))

    return gk(idx, table)


def _mlp2_maxpool(Pg, Q, W2, b2t, M, H, Cout):
    PB = 128

    def body(pg_ref, q_ref, w_ref, b_ref, o_ref):
        q = q_ref[...]
        p = pg_ref[...]
        h = jnp.maximum(p.reshape(PB, KNBR, H) + q[:, None, :], 0.0)
        e = jnp.dot(h.reshape(PB * KNBR, H), w_ref[...],
                    preferred_element_type=jnp.float32) + b_ref[0:1, :]
        e = jnp.maximum(e, 0.0)
        o_ref[...] = jnp.max(e.reshape(PB, KNBR, Cout), axis=1)

    return pl.pallas_call(
        body,
        grid=(M // PB,),
        in_specs=[
            pl.BlockSpec((PB * KNBR, H), lambda i: (i, 0)),
            pl.BlockSpec((PB, H), lambda i: (i, 0)),
            pl.BlockSpec((H, Cout), lambda i: (0, 0)),
            pl.BlockSpec((8, Cout), lambda i: (0, 0)),
        ],
        out_specs=pl.BlockSpec((PB, Cout), lambda i: (i, 0)),
        out_shape=jax.ShapeDtypeStruct((M, Cout), jnp.float32),
    )(Pg, Q, W2, b2t)


def kernel(feats, xyz, W1, b1, W2, b2):
    B, T, N, Cin = feats.shape
    H = W1.shape[1]
    Cout = W2.shape[1]
    M = B * T * N

    feats_flat = feats.reshape(M, Cin)
    xyz_flat = xyz.reshape(M, 3)
    fcol = jnp.broadcast_to(
        jnp.arange(T, dtype=jnp.float32)[None, :, None], (B, T, N)
    ).reshape(M, 1)
    Kp = Cin + 3 + 1
    Kpad = (-Kp) % 8
    X = jnp.concatenate(
        [feats_flat, xyz_flat, fcol, jnp.zeros((M, Kpad), jnp.float32)], axis=1)

    W1a = W1[:Cin]
    W1b = W1[Cin:2 * Cin]
    W1c3 = W1[2 * Cin:2 * Cin + 3]
    W1ct = W1[2 * Cin + 3:2 * Cin + 4] / jnp.maximum(1.0, jnp.float32(WIN))
    zpad = jnp.zeros((Kpad, H), jnp.float32)
    Wp = jnp.concatenate([W1b, W1c3, W1ct, zpad], axis=0)
    Wq = jnp.concatenate([W1a, -W1c3, -W1ct, zpad], axis=0)
    Wcat = jnp.concatenate([Wp, Wq], axis=1)
    bias = jnp.concatenate([jnp.zeros((H,), jnp.float32), b1])
    bias = jnp.broadcast_to(bias[None, :], (8, 2 * H))

    xyz_q = jnp.concatenate(
        [xyz, jnp.zeros((B, T, N, 5), jnp.float32)], axis=-1)
    xyz_c = jnp.swapaxes(xyz_q, 2, 3)

    P, Q = _precompute_pq(X, Wcat, bias, M, H)

    knn = _knn_indices(xyz_q, xyz_c, B, T, N)
    idx = knn.reshape(M * KNBR)

    Pg = _gather_rows(P, idx)

    b2t = jnp.broadcast_to(b2[None, :], (8, Cout))
    out = _mlp2_maxpool(Pg, Q, W2, b2t, M, H, Cout)

    return out.reshape(B, T, N, Cout)


# speedup vs baseline: 46.0160x; 1.2245x over previous
import functools

import jax
import jax.numpy as jnp
from jax import lax
from jax.experimental import pallas as pl
from jax.experimental.pallas import tpu as pltpu
from jax.experimental.pallas import tpu_sc as plsc

KNBR = 8
WIN = 1


def _precompute_pq(X, Wcat, bias, M, H):
    MB = 1024
    Kp = X.shape[1]

    def body(x_ref, w_ref, b_ref, p_ref, q_ref):
        y = jnp.dot(x_ref[...], w_ref[...],
                    preferred_element_type=jnp.float32) + b_ref[0:1, :]
        p_ref[...] = y[:, :H]
        q_ref[...] = y[:, H:]

    return pl.pallas_call(
        body,
        grid=(M // MB,),
        in_specs=[
            pl.BlockSpec((MB, Kp), lambda i: (i, 0)),
            pl.BlockSpec((Kp, 2 * H), lambda i: (0, 0)),
            pl.BlockSpec((8, 2 * H), lambda i: (0, 0)),
        ],
        out_specs=[
            pl.BlockSpec((MB, H), lambda i: (i, 0)),
            pl.BlockSpec((MB, H), lambda i: (i, 0)),
        ],
        out_shape=[
            jax.ShapeDtypeStruct((M, H), jnp.float32),
            jax.ShapeDtypeStruct((M, H), jnp.float32),
        ],
    )(X, Wcat, bias)


def _knn_indices(xyz_q, xyz_c, B, T, N):
    NB = 256
    F = 3
    c0max = T - F
    BIG = 1e30
    IMASK = (1 << 12) - 1
    MAXI = 2**31 - 1

    def body(q_ref, c0_ref, c1_ref, c2_ref, o_ref):
        b = pl.program_id(0)
        t = pl.program_id(1)
        c0 = jnp.minimum(jnp.maximum(t - WIN, 0), c0max)
        q = q_ref[0, 0]
        qn = jnp.sum(q * q, axis=1)
        tiles = []
        for k, c_ref in enumerate((c0_ref, c1_ref, c2_ref)):
            c = c_ref[0, 0]
            cn = jnp.sum(c * c, axis=0)
            dot = jnp.dot(q, c, preferred_element_type=jnp.float32)
            valid = jnp.abs(c0 + k - t) <= WIN
            pen = jnp.where(valid, 0.0, BIG).astype(jnp.float32)
            tiles.append(qn[:, None] + cn[None, :] - 2.0 * dot + pen)
        d2 = jnp.concatenate(tiles, axis=1)
        ji = lax.broadcasted_iota(jnp.int32, (NB, F * N), 1)
        bits = lax.bitcast_convert_type(jnp.maximum(d2, 0.0), jnp.int32)
        key = (bits & ~IMASK) | ji
        picks = []
        for _ in range(KNBR):
            m = jnp.min(key, axis=1, keepdims=True)
            picks.append(m[:, 0] & IMASK)
            key = jnp.where(key == m, MAXI, key)
        base = (b * T + c0) * N
        o_ref[0, 0] = jnp.stack(picks, axis=1) + base

    def cmap(k):
        return lambda b, t, i: (b, jnp.minimum(jnp.maximum(t - WIN, 0), c0max) + k, 0, 0)

    return pl.pallas_call(
        body,
        grid=(B, T, N // NB),
        in_specs=[
            pl.BlockSpec((1, 1, NB, 8), lambda b, t, i: (b, t, i, 0)),
            pl.BlockSpec((1, 1, 8, N), cmap(0)),
            pl.BlockSpec((1, 1, 8, N), cmap(1)),
            pl.BlockSpec((1, 1, 8, N), cmap(2)),
        ],
        out_specs=pl.BlockSpec((1, 1, NB, KNBR), lambda b, t, i: (b, t, i, 0)),
        out_shape=jax.ShapeDtypeStruct((B, T, N, KNBR), jnp.int32),
    )(xyz_q, xyz_c, xyz_c, xyz_c)


def _gather_rows(table, idx):
    NE = idx.shape[0]
    H = table.shape[1]
    info = plsc.get_sparse_core_info()
    NW = info.num_cores * info.num_subcores
    NC = info.num_cores
    per_w = NE // NW
    CH = 128
    nchunk = per_w // CH
    mesh = plsc.VectorSubcoreMesh(core_axis_name="c", subcore_axis_name="s")

    @functools.partial(
        pl.kernel,
        mesh=mesh,
        out_type=jax.ShapeDtypeStruct((NE, H), jnp.float32),
        scratch_types=[
            pltpu.VMEM((CH,), jnp.int32),
            pltpu.VMEM((CH, H), jnp.float32),
            pltpu.SemaphoreType.DMA,
        ],
    )
    def gk(idx_hbm, table_hbm, out_hbm, idx_v, rows_v, sem):
        wid = lax.axis_index("s") * NC + lax.axis_index("c")
        base = wid * per_w

        def chunk(c, carry):
            off = base + c * CH
            pltpu.sync_copy(idx_hbm.at[pl.ds(off, CH)], idx_v)
            pltpu.async_copy(table_hbm.at[idx_v], rows_v, sem).wait()
            pltpu.sync_copy(rows_v, out_hbm.at[pl.ds(off, CH)])
            return carry

        lax.fori_loop(0, nchunk, chunk, jnp.int32(0))

    return gk(idx, table)


def _mlp2_maxpool(Pg, Q, W2, b2t, M, H, Cout):
    PB = 128

    def body(pg_ref, q_ref, w_ref, b_ref, o_ref):
        q = q_ref[...]
        p = pg_ref[...]
        h = jnp.maximum(p.reshape(PB, KNBR, H) + q[:, None, :], 0.0)
        e = jnp.dot(h.reshape(PB * KNBR, H), w_ref[...],
                    preferred_element_type=jnp.float32) + b_ref[0:1, :]
        e = jnp.maximum(e, 0.0)
        o_ref[...] = jnp.max(e.reshape(PB, KNBR, Cout), axis=1)

    return pl.pallas_call(
        body,
        grid=(M // PB,),
        in_specs=[
            pl.BlockSpec((PB * KNBR, H), lambda i: (i, 0)),
            pl.BlockSpec((PB, H), lambda i: (i, 0)),
            pl.BlockSpec((H, Cout), lambda i: (0, 0)),
            pl.BlockSpec((8, Cout), lambda i: (0, 0)),
        ],
        out_specs=pl.BlockSpec((PB, Cout), lambda i: (i, 0)),
        out_shape=jax.ShapeDtypeStruct((M, Cout), jnp.float32),
    )(Pg, Q, W2, b2t)


def kernel(feats, xyz, W1, b1, W2, b2):
    B, T, N, Cin = feats.shape
    H = W1.shape[1]
    Cout = W2.shape[1]
    M = B * T * N

    feats_flat = feats.reshape(M, Cin)
    xyz_flat = xyz.reshape(M, 3)
    fcol = jnp.broadcast_to(
        jnp.arange(T, dtype=jnp.float32)[None, :, None], (B, T, N)
    ).reshape(M, 1)
    Kp = Cin + 3 + 1
    Kpad = (-Kp) % 8
    X = jnp.concatenate(
        [feats_flat, xyz_flat, fcol, jnp.zeros((M, Kpad), jnp.float32)], axis=1)

    W1a = W1[:Cin]
    W1b = W1[Cin:2 * Cin]
    W1c3 = W1[2 * Cin:2 * Cin + 3]
    W1ct = W1[2 * Cin + 3:2 * Cin + 4] / jnp.maximum(1.0, jnp.float32(WIN))
    zpad = jnp.zeros((Kpad, H), jnp.float32)
    Wp = jnp.concatenate([W1b, W1c3, W1ct, zpad], axis=0)
    Wq = jnp.concatenate([W1a, -W1c3, -W1ct, zpad], axis=0)
    Wcat = jnp.concatenate([Wp, Wq], axis=1)
    bias = jnp.concatenate([jnp.zeros((H,), jnp.float32), b1])
    bias = jnp.broadcast_to(bias[None, :], (8, 2 * H))

    xyz_q = jnp.concatenate(
        [xyz, jnp.zeros((B, T, N, 5), jnp.float32)], axis=-1)
    xyz_c = jnp.swapaxes(xyz_q, 2, 3)

    P, Q = _precompute_pq(X, Wcat, bias, M, H)

    knn = _knn_indices(xyz_q, xyz_c, B, T, N)
    idx = knn.reshape(M * KNBR)

    Pg = _gather_rows(P, idx)

    b2t = jnp.broadcast_to(b2[None, :], (8, Cout))
    out = _mlp2_maxpool(Pg, Q, W2, b2t, M, H, Cout)

    return out.reshape(B, T, N, Cout)


# speedup vs baseline: 53.5028x; 1.1627x over previous
import functools

import jax
import jax.numpy as jnp
from jax import lax
from jax.experimental import pallas as pl
from jax.experimental.pallas import tpu as pltpu
from jax.experimental.pallas import tpu_sc as plsc

KNBR = 8
WIN = 1
IBITS = 12
IMASK = (1 << IBITS) - 1
MAXI = 2**31 - 1


def _precompute_pq(X, Wcat, bias, M, H):
    MB = 1024
    Kp = X.shape[1]

    def body(x_ref, w_ref, b_ref, p_ref, q_ref):
        y = jnp.dot(x_ref[...], w_ref[...],
                    preferred_element_type=jnp.float32) + b_ref[0:1, :]
        p_ref[...] = y[:, :H]
        q_ref[...] = y[:, H:]

    return pl.pallas_call(
        body,
        grid=(M // MB,),
        in_specs=[
            pl.BlockSpec((MB, Kp), lambda i: (i, 0)),
            pl.BlockSpec((Kp, 2 * H), lambda i: (0, 0)),
            pl.BlockSpec((8, 2 * H), lambda i: (0, 0)),
        ],
        out_specs=[
            pl.BlockSpec((MB, H), lambda i: (i, 0)),
            pl.BlockSpec((MB, H), lambda i: (i, 0)),
        ],
        out_shape=[
            jax.ShapeDtypeStruct((M, H), jnp.float32),
            jax.ShapeDtypeStruct((M, H), jnp.float32),
        ],
    )(X, Wcat, bias)


def _knn_indices_t(xyz_q, xyz_c, t, B, T, N):
    NB = 256
    t0 = max(0, t - WIN)
    t1 = min(T - 1, t + WIN)
    F = t1 - t0 + 1
    frames = list(range(t0, t1 + 1))
    NMASK = ~(N - 1)

    def body(q_ref, *refs):
        c_refs = refs[:F]
        o_ref = refs[F]
        b = pl.program_id(0)
        q = q_ref[0, 0]
        qn = jnp.sum(q * q, axis=1)
        tiles = []
        for c_ref in c_refs:
            c = c_ref[0, 0]
            cn = jnp.sum(c * c, axis=0)
            dot = jnp.dot(q, c, preferred_element_type=jnp.float32)
            tiles.append(qn[:, None] + cn[None, :] - 2.0 * dot)
        d2 = jnp.concatenate(tiles, axis=1) if F > 1 else tiles[0]
        ji = lax.broadcasted_iota(jnp.int32, (NB, F * N), 1)
        bits = lax.bitcast_convert_type(jnp.maximum(d2, 0.0), jnp.int32)
        key = (bits & ~IMASK) | ji
        picks = []
        for _ in range(KNBR):
            m = jnp.min(key, axis=1, keepdims=True)
            picks.append(m[:, 0] & IMASK)
            key = jnp.where(key == m, MAXI, key)
        j = jnp.stack(picks, axis=1)
        o_ref[0] = j + (j & NMASK) * (B - 1) + (t0 * B + b) * N

    in_specs = [pl.BlockSpec((1, 1, NB, 8), lambda b, i: (t, b, i, 0))]
    for f in frames:
        in_specs.append(
            pl.BlockSpec((1, 1, 8, N), lambda b, i, f=f: (f, b, 0, 0)))

    return pl.pallas_call(
        body,
        grid=(B, N // NB),
        in_specs=in_specs,
        out_specs=pl.BlockSpec((1, NB, KNBR), lambda b, i: (b, i, 0)),
        out_shape=jax.ShapeDtypeStruct((B, N, KNBR), jnp.int32),
    )(xyz_q, *([xyz_c] * F))


def _gather_rows(table, idx):
    NE = idx.shape[0]
    H = table.shape[1]
    info = plsc.get_sparse_core_info()
    NW = info.num_cores * info.num_subcores
    NC = info.num_cores
    per_w = NE // NW
    CH = 128
    nchunk = per_w // CH
    mesh = plsc.VectorSubcoreMesh(core_axis_name="c", subcore_axis_name="s")

    @functools.partial(
        pl.kernel,
        mesh=mesh,
        out_type=jax.ShapeDtypeStruct((NE, H), jnp.float32),
        scratch_types=[
            pltpu.VMEM((CH,), jnp.int32),
            pltpu.VMEM((CH, H), jnp.float32),
            pltpu.SemaphoreType.DMA,
        ],
    )
    def gk(idx_hbm, table_hbm, out_hbm, idx_v, rows_v, sem):
        wid = lax.axis_index("s") * NC + lax.axis_index("c")
        base = wid * per_w

        def chunk(c, carry):
            off = base + c * CH
            pltpu.sync_copy(idx_hbm.at[pl.ds(off, CH)], idx_v)
            pltpu.async_copy(table_hbm.at[idx_v], rows_v, sem).wait()
            pltpu.sync_copy(rows_v, out_hbm.at[pl.ds(off, CH)])
            return carry

        lax.fori_loop(0, nchunk, chunk, jnp.int32(0))

    return gk(idx, table)


def _mlp2_maxpool(Pg, Q, W2, b2t, M, H, Cout):
    PB = 128

    def body(pg_ref, q_ref, w_ref, b_ref, o_ref):
        q = q_ref[...]
        p = pg_ref[...]
        h = jnp.maximum(p.reshape(PB, KNBR, H) + q[:, None, :], 0.0)
        e = jnp.dot(h.reshape(PB * KNBR, H), w_ref[...],
                    preferred_element_type=jnp.float32) + b_ref[0:1, :]
        e = jnp.maximum(e, 0.0)
        o_ref[...] = jnp.max(e.reshape(PB, KNBR, Cout), axis=1)

    return pl.pallas_call(
        body,
        grid=(M // PB,),
        in_specs=[
            pl.BlockSpec((PB * KNBR, H), lambda i: (i, 0)),
            pl.BlockSpec((PB, H), lambda i: (i, 0)),
            pl.BlockSpec((H, Cout), lambda i: (0, 0)),
            pl.BlockSpec((8, Cout), lambda i: (0, 0)),
        ],
        out_specs=pl.BlockSpec((PB, Cout), lambda i: (i, 0)),
        out_shape=jax.ShapeDtypeStruct((M, Cout), jnp.float32),
    )(Pg, Q, W2, b2t)


def kernel(feats, xyz, W1, b1, W2, b2):
    B, T, N, Cin = feats.shape
    H = W1.shape[1]
    Cout = W2.shape[1]
    M = T * B * N

    feats_t = jnp.swapaxes(feats, 0, 1)
    xyz_t = jnp.swapaxes(xyz, 0, 1)
    fcol = jnp.broadcast_to(
        jnp.arange(T, dtype=jnp.float32)[:, None, None], (T, B, N)
    ).reshape(M, 1)
    Kp = Cin + 3 + 1
    Kpad = (-Kp) % 8
    X = jnp.concatenate(
        [feats_t.reshape(M, Cin), xyz_t.reshape(M, 3), fcol,
         jnp.zeros((M, Kpad), jnp.float32)], axis=1)

    W1a = W1[:Cin]
    W1b = W1[Cin:2 * Cin]
    W1c3 = W1[2 * Cin:2 * Cin + 3]
    W1ct = W1[2 * Cin + 3:2 * Cin + 4] / jnp.maximum(1.0, jnp.float32(WIN))
    zpad = jnp.zeros((Kpad, H), jnp.float32)
    Wp = jnp.concatenate([W1b, W1c3, W1ct, zpad], axis=0)
    Wq = jnp.concatenate([W1a, -W1c3, -W1ct, zpad], axis=0)
    Wcat = jnp.concatenate([Wp, Wq], axis=1)
    bias = jnp.concatenate([jnp.zeros((H,), jnp.float32), b1])
    bias = jnp.broadcast_to(bias[None, :], (8, 2 * H))
    b2t = jnp.broadcast_to(b2[None, :], (8, Cout))

    xyz_q = jnp.concatenate(
        [xyz_t, jnp.zeros((T, B, N, 5), jnp.float32)], axis=-1)
    xyz_c = jnp.swapaxes(xyz_q, 2, 3)

    P, Q = _precompute_pq(X, Wcat, bias, M, H)

    outs = []
    for t in range(T):
        knn = _knn_indices_t(xyz_q, xyz_c, t, B, T, N)
        idx = knn.reshape(B * N * KNBR)
        Pg = _gather_rows(P, idx)
        Qt = lax.dynamic_slice_in_dim(Q, t * B * N, B * N, 0)
        out_t = _mlp2_maxpool(Pg, Qt, W2, b2t, B * N, H, Cout)
        outs.append(out_t.reshape(B, N, Cout))

    return jnp.stack(outs, axis=1)
